# baseline (speedup 1.0000x reference)
import functools

import jax
import jax.numpy as jnp
import numpy as np
from jax import lax
from jax.experimental import pallas as pl
from jax.experimental.pallas import tpu as pltpu
from jax.experimental.pallas import tpu_sc as plsc

F_ = 26
V = 100000
K = 16
DFF = 128
NLAYERS = 5

_NC = 2
_NS = 16
_NW = _NC * _NS
_CH = 128

_BN = float(1.0 / np.sqrt(1.0 + 1e-5))
_G = 16


def _sc_gather(tab2, tab1, idx3d, n_per, n_ch):
    N = _NW * n_per
    mesh = plsc.VectorSubcoreMesh(core_axis_name="c", subcore_axis_name="s")

    @functools.partial(
        pl.kernel,
        mesh=mesh,
        compiler_params=pltpu.CompilerParams(use_tc_tiling_on_sc=False),
        out_type=(
            jax.ShapeDtypeStruct((N, K), jnp.float32),
            jax.ShapeDtypeStruct((N, 1), jnp.float32),
        ),
        scratch_types=[
            pltpu.VMEM((n_ch, _CH), jnp.int32),
            pltpu.VMEM((n_per, K), jnp.float32),
            pltpu.VMEM((n_per, 1), jnp.float32),
            pltpu.SemaphoreType.DMA,
            pltpu.SemaphoreType.DMA,
        ],
    )
    def gather_kernel(idx_hbm, tab2_hbm, tab1_hbm, out2_hbm, out1_hbm,
                      idx_v, rows_v, w1_v, sem2, sem1):
        wid = lax.axis_index("s") * _NC + lax.axis_index("c")
        pltpu.sync_copy(idx_hbm.at[wid], idx_v)

        def body(c, carry):
            pltpu.async_copy(
                tab2_hbm.at[idx_v.at[c]],
                rows_v.at[pl.ds(c * _CH, _CH)], sem2).wait()
            pltpu.async_copy(
                tab1_hbm.at[idx_v.at[c]],
                w1_v.at[pl.ds(c * _CH, _CH)], sem1).wait()
            return carry

        lax.fori_loop(0, n_ch, body, 0)
        base = wid * n_per
        pltpu.sync_copy(rows_v, out2_hbm.at[pl.ds(base, n_per)])
        pltpu.sync_copy(w1_v, out1_hbm.at[pl.ds(base, n_per)])

    return gather_kernel(idx3d, tab2, tab1)


def _dense_body(G, R,
                rows_r, xv_r, w1_r, pet_r, mask_r, sel_r,
                wqkv_r, bqkv_r, wo_r, bo_r,
                n1a_r, n1b_r, n2a_r, n2b_r,
                fw1_r, fb1_r, fw2_r, fb2_r,
                na_r, nb_r,
                m0t_r, m2t_r, m1t_r, hb_r,
                c1_r, cb1_r, c2_r, cb2_r,
                out_r):
    f32 = jnp.float32

    def nrm(x, a, b):
        m = jnp.mean(x, axis=-1, keepdims=True)
        d = x - m
        var = jnp.sum(d * d, axis=-1, keepdims=True) * (1.0 / (K - 1))
        return a * d / (jnp.sqrt(var) + 1e-6) + b

    rows = rows_r[...]
    xv = xv_r[...]
    w2 = rows * xv
    sel = sel_r[...]
    ssum = jnp.dot(sel, w2, preferred_element_type=f32)
    sqsum = jnp.dot(sel, w2 * w2, preferred_element_type=f32)
    second = 0.5 * (ssum * ssum - sqsum)
    x = w2 * float(np.sqrt(K)) + pet_r[...]
    mask = mask_r[...]
    for l in range(NLAYERS):
        x2 = nrm(x, n1a_r[l], n1b_r[l])
        qkv = jnp.dot(x2, wqkv_r[l], preferred_element_type=f32) + bqkv_r[l]
        q = qkv[:, 0:K]
        k = qkv[:, K:2 * K]
        v = qkv[:, 2 * K:3 * K]
        scores = lax.dot_general(q, k, (((1,), (1,)), ((), ())),
                                 preferred_element_type=f32)
        scores = scores * mask
        att = jnp.dot(scores, v, preferred_element_type=f32)
        att = jnp.dot(att, wo_r[l], preferred_element_type=f32) + bo_r[l]
        x = x + att
        x2 = nrm(x, n2a_r[l], n2b_r[l])
        h = jnp.dot(x2, fw1_r[l], preferred_element_type=f32) + fb1_r[l]
        h = jnp.maximum(h * _BN, 0.0)
        ff = jnp.dot(h, fw2_r[l], preferred_element_type=f32) + fb2_r[l]
        x = x + ff
    x = nrm(x, na_r[...], nb_r[...])
    first_col = w1_r[...] * xv
    a0 = first_col * m0t_r[...]
    m0 = jnp.dot(sel, a0, preferred_element_type=f32)
    m1 = jnp.dot(second, m1t_r[...], preferred_element_type=f32)
    a2c = [jnp.sum(x * m2t_r[o * R:(o + 1) * R, :], axis=-1, keepdims=True)
           for o in range(4)]
    a2 = jnp.concatenate(a2c, axis=1)
    m2 = jnp.dot(sel, a2, preferred_element_type=f32)
    hb = hb_r[...]
    c1 = c1_r[...]
    h = (jnp.dot(m0 + hb[0:1], c1[0:4], preferred_element_type=f32)
         + jnp.dot(m1 + hb[1:2], c1[4:8], preferred_element_type=f32)
         + jnp.dot(m2 + hb[2:3], c1[8:12], preferred_element_type=f32)
         + cb1_r[...])
    h = jnp.maximum(h * _BN, 0.0)
    out_r[...] = jnp.dot(h, c2_r[...], preferred_element_type=f32) + cb2_r[...]


def _dense_forward(rows2d, xv_col, w1_col, params, pe, G):
    N = rows2d.shape[0]
    B = N // F_
    R = G * F_
    f32 = jnp.float32
    enc = params['enc']
    st = lambda key: jnp.stack([p[key] for p in enc])
    wqkv = jnp.concatenate([jnp.swapaxes(st('wq'), 1, 2),
                            jnp.swapaxes(st('wk'), 1, 2),
                            jnp.swapaxes(st('wv'), 1, 2)], axis=2)
    bqkv = jnp.concatenate([st('bq'), st('bk'), st('bv')], axis=1)[:, None, :]
    wo = jnp.swapaxes(st('wo'), 1, 2)
    bo = st('bo')[:, None, :]
    n1a = st('n1_a')[:, None, :]
    n1b = st('n1_b')[:, None, :]
    n2a = st('n2_a')[:, None, :]
    n2b = st('n2_b')[:, None, :]
    fw1 = jnp.swapaxes(st('ffw1'), 1, 2)
    fb1 = st('ffb1')[:, None, :]
    fw2 = jnp.swapaxes(st('ffw2'), 1, 2)
    fb2 = st('ffb2')[:, None, :]
    na = params['norm2_a'][None, :]
    nb = params['norm2_b'][None, :]
    pet = jnp.tile(pe, (G, 1))
    seg = np.arange(R) // F_
    mask = jnp.asarray((seg[:, None] == seg[None, :]).astype(np.float32)
                       * (1.0 / np.sqrt(K)), dtype=f32)
    sel = jnp.asarray((np.arange(G)[:, None] == seg[None, :]).astype(np.float32))
    m0t = jnp.tile(params['m0_w'].T, (G, 1))
    m2t = jnp.tile(params['m2_w'].reshape(4, F_, K), (1, G, 1)).reshape(4 * R, K)
    m1t = params['m1_w'].T
    hb = jnp.stack([params['m0_b'], params['m1_b'], params['m2_b']])
    c1 = params['cls_w1'].T
    cb1 = params['cls_b1'][None, :]
    c2 = params['cls_w2'].T
    cb2 = params['cls_b2'][None, :]

    nblk = B // G
    cst = lambda *shape: pl.BlockSpec(shape, lambda i: (0,) * len(shape))
    grid_spec = pl.GridSpec(
        grid=(nblk,),
        in_specs=[
            pl.BlockSpec((R, K), lambda i: (i, 0)),
            pl.BlockSpec((R, 1), lambda i: (i, 0)),
            pl.BlockSpec((R, 1), lambda i: (i, 0)),
            cst(R, K), cst(R, R), cst(G, R),
            cst(NLAYERS, K, 3 * K), cst(NLAYERS, 1, 3 * K),
            cst(NLAYERS, K, K), cst(NLAYERS, 1, K),
            cst(NLAYERS, 1, K), cst(NLAYERS, 1, K),
            cst(NLAYERS, 1, K), cst(NLAYERS, 1, K),
            cst(NLAYERS, K, DFF), cst(NLAYERS, 1, DFF),
            cst(NLAYERS, DFF, K), cst(NLAYERS, 1, K),
            cst(1, K), cst(1, K),
            cst(R, 4), cst(4 * R, K), cst(K, 4), cst(3, 4),
            cst(12, DFF), cst(1, DFF), cst(DFF, 2), cst(1, 2),
        ],
        out_specs=pl.BlockSpec((G, 2), lambda i: (i, 0)),
    )
    fn = pl.pallas_call(
        functools.partial(_dense_body, G, R),
        grid_spec=grid_spec,
        out_shape=jax.ShapeDtypeStruct((B, 2), f32),
    )
    return fn(rows2d, xv_col, w1_col, pet, mask, sel,
              wqkv, bqkv, wo, bo, n1a, n1b, n2a, n2b,
              fw1, fb1, fw2, fb2, na, nb,
              m0t, m2t, m1t, hb, c1, cb1, c2, cb2)


def kernel(Xi, Xv, params, pe):
    B = Xi.shape[0]
    N = B * F_

    idx = Xi[..., 0].astype(jnp.int32) + (jnp.arange(F_, dtype=jnp.int32) * V)[None, :]
    n_per = N // _NW
    n_ch = n_per // _CH
    idx3d = idx.reshape(_NW, n_ch, _CH)
    tab2 = params['fm_w2'].reshape(F_ * V, K)
    tab1 = params['fm_w1'].reshape(F_ * V, 1)
    rows2, rows1 = _sc_gather(tab2, tab1, idx3d, n_per, n_ch)

    xv_col = Xv.reshape(N, 1)
    return _dense_forward(rows2, xv_col, rows1, params, pe, _G)


# speedup vs baseline: 1.9075x; 1.9075x over previous
import functools

import jax
import jax.numpy as jnp
import numpy as np
from jax import lax
from jax.experimental import pallas as pl
from jax.experimental.pallas import tpu as pltpu
from jax.experimental.pallas import tpu_sc as plsc

F_ = 26
V = 100000
K = 16
DFF = 128
NLAYERS = 5

_NC = 2
_NS = 16
_NW = _NC * _NS
_CH = 128

_BN = float(1.0 / np.sqrt(1.0 + 1e-5))
_G = 16


def _sc_gather(tab2, tab1, idx3d, idx8_3d, n_per, n_ch):
    N = _NW * n_per
    mesh = plsc.VectorSubcoreMesh(core_axis_name="c", subcore_axis_name="s")

    @functools.partial(
        pl.kernel,
        mesh=mesh,
        compiler_params=pltpu.CompilerParams(use_tc_tiling_on_sc=False),
        out_type=(
            jax.ShapeDtypeStruct((N, 128), jnp.float32),
            jax.ShapeDtypeStruct((N,), jnp.float32),
        ),
        scratch_types=[
            pltpu.VMEM((n_ch, _CH), jnp.int32),
            pltpu.VMEM((n_ch, _CH), jnp.int32),
            pltpu.VMEM((_CH, 128), jnp.float32),
            pltpu.VMEM((n_per,), jnp.float32),
            pltpu.SemaphoreType.DMA,
            pltpu.SemaphoreType.DMA,
        ],
    )
    def gather_kernel(idx_hbm, idx8_hbm, tab2_hbm, tab1_hbm, out2_hbm, out1_hbm,
                      idx_v, idx8_v, chunk_v, w1_v, sem2, sem1):
        wid = lax.axis_index("s") * _NC + lax.axis_index("c")
        pltpu.sync_copy(idx_hbm.at[wid], idx_v)
        pltpu.sync_copy(idx8_hbm.at[wid], idx8_v)
        base = wid * n_per

        def body(c, carry):
            pltpu.async_copy(
                tab2_hbm.at[idx8_v.at[c]], chunk_v, sem2).wait()
            pltpu.async_copy(
                tab1_hbm.at[idx_v.at[c]],
                w1_v.at[pl.ds(c * _CH, _CH)], sem1).wait()
            pltpu.sync_copy(chunk_v, out2_hbm.at[pl.ds(base + c * _CH, _CH)])
            return carry

        lax.fori_loop(0, n_ch, body, 0)
        pltpu.sync_copy(w1_v, out1_hbm.at[pl.ds(base, n_per)])

    return gather_kernel(idx3d, idx8_3d, tab2, tab1)


def _dense_body(G, R,
                rows_r, off_r, xv_r, w1_r, pet_r, mask_r, sel_r,
                wqkv_r, bqkv_r, wo_r, bo_r,
                n1a_r, n1b_r, n2a_r, n2b_r,
                fw1_r, fb1_r, fw2_r, fb2_r,
                na_r, nb_r,
                m0t_r, m2t_r, m1t_r, hb_r,
                c1_r, cb1_r, c2_r, cb2_r,
                out_r):
    f32 = jnp.float32

    def nrm(x, a, b):
        m = jnp.mean(x, axis=-1, keepdims=True)
        d = x - m
        var = jnp.sum(d * d, axis=-1, keepdims=True) * (1.0 / (K - 1))
        return a * d / (jnp.sqrt(var) + 1e-6) + b

    packed = rows_r[...]
    off = off_r[...]
    rows = jnp.zeros((R, K), f32)
    for j in range(8):
        rows = rows + jnp.where(off == float(j),
                                packed[:, j * K:(j + 1) * K], 0.0)
    xv = xv_r[...]
    w2 = rows * xv
    sel = sel_r[...]
    ssum = jnp.dot(sel, w2, preferred_element_type=f32)
    sqsum = jnp.dot(sel, w2 * w2, preferred_element_type=f32)
    second = 0.5 * (ssum * ssum - sqsum)
    x = w2 * float(np.sqrt(K)) + pet_r[...]
    mask = mask_r[...]
    for l in range(NLAYERS):
        x2 = nrm(x, n1a_r[l], n1b_r[l])
        qkv = jnp.dot(x2, wqkv_r[l], preferred_element_type=f32) + bqkv_r[l]
        q = qkv[:, 0:K]
        k = qkv[:, K:2 * K]
        v = qkv[:, 2 * K:3 * K]
        scores = lax.dot_general(q, k, (((1,), (1,)), ((), ())),
                                 preferred_element_type=f32)
        scores = scores * mask
        att = jnp.dot(scores, v, preferred_element_type=f32)
        att = jnp.dot(att, wo_r[l], preferred_element_type=f32) + bo_r[l]
        x = x + att
        x2 = nrm(x, n2a_r[l], n2b_r[l])
        h = jnp.dot(x2, fw1_r[l], preferred_element_type=f32) + fb1_r[l]
        h = jnp.maximum(h * _BN, 0.0)
        ff = jnp.dot(h, fw2_r[l], preferred_element_type=f32) + fb2_r[l]
        x = x + ff
    x = nrm(x, na_r[...], nb_r[...])
    first_col = w1_r[...] * xv
    a0 = first_col * m0t_r[...]
    m0 = jnp.dot(sel, a0, preferred_element_type=f32)
    m1 = jnp.dot(second, m1t_r[...], preferred_element_type=f32)
    a2c = [jnp.sum(x * m2t_r[o * R:(o + 1) * R, :], axis=-1, keepdims=True)
           for o in range(4)]
    a2 = jnp.concatenate(a2c, axis=1)
    m2 = jnp.dot(sel, a2, preferred_element_type=f32)
    hb = hb_r[...]
    c1 = c1_r[...]
    h = (jnp.dot(m0 + hb[0:1], c1[0:4], preferred_element_type=f32)
         + jnp.dot(m1 + hb[1:2], c1[4:8], preferred_element_type=f32)
         + jnp.dot(m2 + hb[2:3], c1[8:12], preferred_element_type=f32)
         + cb1_r[...])
    h = jnp.maximum(h * _BN, 0.0)
    out_r[...] = jnp.dot(h, c2_r[...], preferred_element_type=f32) + cb2_r[...]


def _dense_forward(packed, off_col, xv_col, w1_col, params, pe, G):
    N = packed.shape[0]
    B = N // F_
    R = G * F_
    f32 = jnp.float32
    enc = params['enc']
    st = lambda key: jnp.stack([p[key] for p in enc])
    wqkv = jnp.concatenate([jnp.swapaxes(st('wq'), 1, 2),
                            jnp.swapaxes(st('wk'), 1, 2),
                            jnp.swapaxes(st('wv'), 1, 2)], axis=2)
    bqkv = jnp.concatenate([st('bq'), st('bk'), st('bv')], axis=1)[:, None, :]
    wo = jnp.swapaxes(st('wo'), 1, 2)
    bo = st('bo')[:, None, :]
    n1a = st('n1_a')[:, None, :]
    n1b = st('n1_b')[:, None, :]
    n2a = st('n2_a')[:, None, :]
    n2b = st('n2_b')[:, None, :]
    fw1 = jnp.swapaxes(st('ffw1'), 1, 2)
    fb1 = st('ffb1')[:, None, :]
    fw2 = jnp.swapaxes(st('ffw2'), 1, 2)
    fb2 = st('ffb2')[:, None, :]
    na = params['norm2_a'][None, :]
    nb = params['norm2_b'][None, :]
    pet = jnp.tile(pe, (G, 1))
    seg = np.arange(R) // F_
    mask = jnp.asarray((seg[:, None] == seg[None, :]).astype(np.float32)
                       * (1.0 / np.sqrt(K)), dtype=f32)
    sel = jnp.asarray((np.arange(G)[:, None] == seg[None, :]).astype(np.float32))
    m0t = jnp.tile(params['m0_w'].T, (G, 1))
    m2t = jnp.tile(params['m2_w'].reshape(4, F_, K), (1, G, 1)).reshape(4 * R, K)
    m1t = params['m1_w'].T
    hb = jnp.stack([params['m0_b'], params['m1_b'], params['m2_b']])
    c1 = params['cls_w1'].T
    cb1 = params['cls_b1'][None, :]
    c2 = params['cls_w2'].T
    cb2 = params['cls_b2'][None, :]

    nblk = B // G
    cst = lambda *shape: pl.BlockSpec(shape, lambda i: (0,) * len(shape))
    grid_spec = pl.GridSpec(
        grid=(nblk,),
        in_specs=[
            pl.BlockSpec((R, 128), lambda i: (i, 0)),
            pl.BlockSpec((R, 1), lambda i: (i, 0)),
            pl.BlockSpec((R, 1), lambda i: (i, 0)),
            pl.BlockSpec((R, 1), lambda i: (i, 0)),
            cst(R, K), cst(R, R), cst(G, R),
            cst(NLAYERS, K, 3 * K), cst(NLAYERS, 1, 3 * K),
            cst(NLAYERS, K, K), cst(NLAYERS, 1, K),
            cst(NLAYERS, 1, K), cst(NLAYERS, 1, K),
            cst(NLAYERS, 1, K), cst(NLAYERS, 1, K),
            cst(NLAYERS, K, DFF), cst(NLAYERS, 1, DFF),
            cst(NLAYERS, DFF, K), cst(NLAYERS, 1, K),
            cst(1, K), cst(1, K),
            cst(R, 4), cst(4 * R, K), cst(K, 4), cst(3, 4),
            cst(12, DFF), cst(1, DFF), cst(DFF, 2), cst(1, 2),
        ],
        out_specs=pl.BlockSpec((G, 2), lambda i: (i, 0)),
    )
    fn = pl.pallas_call(
        functools.partial(_dense_body, G, R),
        grid_spec=grid_spec,
        out_shape=jax.ShapeDtypeStruct((B, 2), f32),
    )
    return fn(packed, off_col, xv_col, w1_col, pet, mask, sel,
              wqkv, bqkv, wo, bo, n1a, n1b, n2a, n2b,
              fw1, fb1, fw2, fb2, na, nb,
              m0t, m2t, m1t, hb, c1, cb1, c2, cb2)


def kernel(Xi, Xv, params, pe):
    B = Xi.shape[0]
    N = B * F_

    idx = Xi[..., 0].astype(jnp.int32) + (jnp.arange(F_, dtype=jnp.int32) * V)[None, :]
    n_per = N // _NW
    n_ch = n_per // _CH
    idx3d = idx.reshape(_NW, n_ch, _CH)
    idx8_3d = lax.shift_right_logical(idx3d, 3)
    tab2 = params['fm_w2'].reshape(F_ * V // 8, 128)
    tab1 = params['fm_w1'].reshape(F_ * V)
    packed, w1flat = _sc_gather(tab2, tab1, idx3d, idx8_3d, n_per, n_ch)

    off_col = jnp.remainder(idx, 8).astype(jnp.float32).reshape(N, 1)
    xv_col = Xv.reshape(N, 1)
    w1_col = w1flat.reshape(N, 1)
    return _dense_forward(packed, off_col, xv_col, w1_col, params, pe, _G)


# speedup vs baseline: 2.8866x; 1.5133x over previous
import functools

import jax
import jax.numpy as jnp
import numpy as np
from jax import lax
from jax.experimental import pallas as pl
from jax.experimental.pallas import tpu as pltpu
from jax.experimental.pallas import tpu_sc as plsc

F_ = 26
V = 100000
K = 16
DFF = 128
NLAYERS = 5

_NC = 2
_NS = 16
_NW = _NC * _NS
_CH = 128

_BN = float(1.0 / np.sqrt(1.0 + 1e-5))
_G = 64
_AG = 16


def _sc_gather(tab2, tab1, idx3d, idx8_3d, n_per, n_ch):
    N = _NW * n_per
    mesh = plsc.VectorSubcoreMesh(core_axis_name="c", subcore_axis_name="s")

    @functools.partial(
        pl.kernel,
        mesh=mesh,
        compiler_params=pltpu.CompilerParams(use_tc_tiling_on_sc=False),
        out_type=(
            jax.ShapeDtypeStruct((N, 128), jnp.float32),
            jax.ShapeDtypeStruct((N,), jnp.float32),
        ),
        scratch_types=[
            pltpu.VMEM((n_ch, _CH), jnp.int32),
            pltpu.VMEM((n_ch, _CH), jnp.int32),
            pltpu.VMEM((_CH, 128), jnp.float32),
            pltpu.VMEM((n_per,), jnp.float32),
            pltpu.SemaphoreType.DMA,
            pltpu.SemaphoreType.DMA,
        ],
    )
    def gather_kernel(idx_hbm, idx8_hbm, tab2_hbm, tab1_hbm, out2_hbm, out1_hbm,
                      idx_v, idx8_v, chunk_v, w1_v, sem2, sem1):
        wid = lax.axis_index("s") * _NC + lax.axis_index("c")
        pltpu.sync_copy(idx_hbm.at[wid], idx_v)
        pltpu.sync_copy(idx8_hbm.at[wid], idx8_v)
        base = wid * n_per

        def body(c, carry):
            pltpu.async_copy(
                tab2_hbm.at[idx8_v.at[c]], chunk_v, sem2).wait()
            pltpu.async_copy(
                tab1_hbm.at[idx_v.at[c]],
                w1_v.at[pl.ds(c * _CH, _CH)], sem1).wait()
            pltpu.sync_copy(chunk_v, out2_hbm.at[pl.ds(base + c * _CH, _CH)])
            return carry

        lax.fori_loop(0, n_ch, body, 0)
        pltpu.sync_copy(w1_v, out1_hbm.at[pl.ds(base, n_per)])

    return gather_kernel(idx3d, idx8_3d, tab2, tab1)


def _dense_body(G, R,
                rows_r, off_r, xv_r, w1_r, pet_r, mask_r, sel_r,
                wqkv_r, bqkv_r, wo_r, bo_r,
                n1a_r, n1b_r, n2a_r, n2b_r,
                fw1_r, fb1_r, fw2_r, fb2_r,
                na_r, nb_r,
                m0t_r, m2t_r, m1t_r, hb_r,
                c1_r, cb1_r, c2_r, cb2_r,
                out_r):
    f32 = jnp.float32

    def nrm(x, a, b):
        m = jnp.mean(x, axis=-1, keepdims=True)
        d = x - m
        var = jnp.sum(d * d, axis=-1, keepdims=True) * (1.0 / (K - 1))
        return a * d / (jnp.sqrt(var) + 1e-6) + b

    packed = rows_r[...]
    off = off_r[...]
    rows = jnp.zeros((R, K), f32)
    for j in range(8):
        rows = rows + jnp.where(off == float(j),
                                packed[:, j * K:(j + 1) * K], 0.0)
    xv = xv_r[...]
    w2 = rows * xv
    sel = sel_r[...]
    ssum = jnp.dot(sel, w2, preferred_element_type=f32)
    sqsum = jnp.dot(sel, w2 * w2, preferred_element_type=f32)
    second = 0.5 * (ssum * ssum - sqsum)
    x = w2 * float(np.sqrt(K)) + pet_r[...]
    mask = mask_r[...]
    for l in range(NLAYERS):
        x2 = nrm(x, n1a_r[l], n1b_r[l])
        qkv = jnp.dot(x2, wqkv_r[l], preferred_element_type=f32) + bqkv_r[l]
        q = qkv[:, 0:K]
        k = qkv[:, K:2 * K]
        v = qkv[:, 2 * K:3 * K]
        rc = _AG * F_
        attc = []
        for ci in range(R // rc):
            qc = q[ci * rc:(ci + 1) * rc, :]
            kc = k[ci * rc:(ci + 1) * rc, :]
            vc = v[ci * rc:(ci + 1) * rc, :]
            s = lax.dot_general(qc, kc, (((1,), (1,)), ((), ())),
                                preferred_element_type=f32)
            s = s * mask
            attc.append(jnp.dot(s, vc, preferred_element_type=f32))
        att = attc[0] if len(attc) == 1 else jnp.concatenate(attc, axis=0)
        att = jnp.dot(att, wo_r[l], preferred_element_type=f32) + bo_r[l]
        x = x + att
        x2 = nrm(x, n2a_r[l], n2b_r[l])
        h = jnp.dot(x2, fw1_r[l], preferred_element_type=f32) + fb1_r[l]
        h = jnp.maximum(h * _BN, 0.0)
        ff = jnp.dot(h, fw2_r[l], preferred_element_type=f32) + fb2_r[l]
        x = x + ff
    x = nrm(x, na_r[...], nb_r[...])
    first_col = w1_r[...] * xv
    a0 = first_col * m0t_r[...]
    m0 = jnp.dot(sel, a0, preferred_element_type=f32)
    m1 = jnp.dot(second, m1t_r[...], preferred_element_type=f32)
    a2c = [jnp.sum(x * m2t_r[o * R:(o + 1) * R, :], axis=-1, keepdims=True)
           for o in range(4)]
    a2 = jnp.concatenate(a2c, axis=1)
    m2 = jnp.dot(sel, a2, preferred_element_type=f32)
    hb = hb_r[...]
    c1 = c1_r[...]
    h = (jnp.dot(m0 + hb[0:1], c1[0:4], preferred_element_type=f32)
         + jnp.dot(m1 + hb[1:2], c1[4:8], preferred_element_type=f32)
         + jnp.dot(m2 + hb[2:3], c1[8:12], preferred_element_type=f32)
         + cb1_r[...])
    h = jnp.maximum(h * _BN, 0.0)
    out_r[...] = jnp.dot(h, c2_r[...], preferred_element_type=f32) + cb2_r[...]


def _dense_forward(packed, off_col, xv_col, w1_col, params, pe, G):
    N = packed.shape[0]
    B = N // F_
    R = G * F_
    f32 = jnp.float32
    enc = params['enc']
    st = lambda key: jnp.stack([p[key] for p in enc])
    wqkv = jnp.concatenate([jnp.swapaxes(st('wq'), 1, 2),
                            jnp.swapaxes(st('wk'), 1, 2),
                            jnp.swapaxes(st('wv'), 1, 2)], axis=2)
    bqkv = jnp.concatenate([st('bq'), st('bk'), st('bv')], axis=1)[:, None, :]
    wo = jnp.swapaxes(st('wo'), 1, 2)
    bo = st('bo')[:, None, :]
    n1a = st('n1_a')[:, None, :]
    n1b = st('n1_b')[:, None, :]
    n2a = st('n2_a')[:, None, :]
    n2b = st('n2_b')[:, None, :]
    fw1 = jnp.swapaxes(st('ffw1'), 1, 2)
    fb1 = st('ffb1')[:, None, :]
    fw2 = jnp.swapaxes(st('ffw2'), 1, 2)
    fb2 = st('ffb2')[:, None, :]
    na = params['norm2_a'][None, :]
    nb = params['norm2_b'][None, :]
    pet = jnp.tile(pe, (G, 1))
    rc = _AG * F_
    segc = np.arange(rc) // F_
    mask = jnp.asarray((segc[:, None] == segc[None, :]).astype(np.float32)
                       * (1.0 / np.sqrt(K)), dtype=f32)
    seg = np.arange(R) // F_
    sel = jnp.asarray((np.arange(G)[:, None] == seg[None, :]).astype(np.float32))
    m0t = jnp.tile(params['m0_w'].T, (G, 1))
    m2t = jnp.tile(params['m2_w'].reshape(4, F_, K), (1, G, 1)).reshape(4 * R, K)
    m1t = params['m1_w'].T
    hb = jnp.stack([params['m0_b'], params['m1_b'], params['m2_b']])
    c1 = params['cls_w1'].T
    cb1 = params['cls_b1'][None, :]
    c2 = params['cls_w2'].T
    cb2 = params['cls_b2'][None, :]

    nblk = B // G
    cst = lambda *shape: pl.BlockSpec(shape, lambda i: (0,) * len(shape))
    grid_spec = pl.GridSpec(
        grid=(nblk,),
        in_specs=[
            pl.BlockSpec((R, 128), lambda i: (i, 0)),
            pl.BlockSpec((R, 1), lambda i: (i, 0)),
            pl.BlockSpec((R, 1), lambda i: (i, 0)),
            pl.BlockSpec((R, 1), lambda i: (i, 0)),
            cst(R, K), cst(_AG * F_, _AG * F_), cst(G, R),
            cst(NLAYERS, K, 3 * K), cst(NLAYERS, 1, 3 * K),
            cst(NLAYERS, K, K), cst(NLAYERS, 1, K),
            cst(NLAYERS, 1, K), cst(NLAYERS, 1, K),
            cst(NLAYERS, 1, K), cst(NLAYERS, 1, K),
            cst(NLAYERS, K, DFF), cst(NLAYERS, 1, DFF),
            cst(NLAYERS, DFF, K), cst(NLAYERS, 1, K),
            cst(1, K), cst(1, K),
            cst(R, 4), cst(4 * R, K), cst(K, 4), cst(3, 4),
            cst(12, DFF), cst(1, DFF), cst(DFF, 2), cst(1, 2),
        ],
        out_specs=pl.BlockSpec((G, 2), lambda i: (i, 0)),
    )
    fn = pl.pallas_call(
        functools.partial(_dense_body, G, R),
        grid_spec=grid_spec,
        out_shape=jax.ShapeDtypeStruct((B, 2), f32),
    )
    return fn(packed, off_col, xv_col, w1_col, pet, mask, sel,
              wqkv, bqkv, wo, bo, n1a, n1b, n2a, n2b,
              fw1, fb1, fw2, fb2, na, nb,
              m0t, m2t, m1t, hb, c1, cb1, c2, cb2)


def kernel(Xi, Xv, params, pe):
    B = Xi.shape[0]
    N = B * F_

    idx = Xi[..., 0].astype(jnp.int32) + (jnp.arange(F_, dtype=jnp.int32) * V)[None, :]
    n_per = N // _NW
    n_ch = n_per // _CH
    idx3d = idx.reshape(_NW, n_ch, _CH)
    idx8_3d = lax.shift_right_logical(idx3d, 3)
    tab2 = params['fm_w2'].reshape(F_ * V // 8, 128)
    tab1 = params['fm_w1'].reshape(F_ * V)
    packed, w1flat = _sc_gather(tab2, tab1, idx3d, idx8_3d, n_per, n_ch)

    off_col = jnp.remainder(idx, 8).astype(jnp.float32).reshape(N, 1)
    xv_col = Xv.reshape(N, 1)
    w1_col = w1flat.reshape(N, 1)
    return _dense_forward(packed, off_col, xv_col, w1_col, params, pe, _G)
